# baseline (speedup 1.0000x reference)
import functools

import jax
import jax.numpy as jnp
from jax import lax
from jax.experimental import pallas as pl
from jax.experimental.pallas import tpu as pltpu
from jax.experimental.pallas import tpu_sc as plsc

_L = 16


def _make_sc_embed(N, D, P, S):
    info = plsc.get_sparse_core_info()
    NC, NS = info.num_cores, info.num_subcores
    NW = NC * NS
    assert N % NW == 0
    tok_w = N // NW
    CH = 128
    assert tok_w % (2 * CH) == 0
    n_chunks = tok_w // CH

    mesh = plsc.VectorSubcoreMesh(core_axis_name="c", subcore_axis_name="s")

    @functools.partial(
        pl.kernel,
        mesh=mesh,
        compiler_params=pltpu.CompilerParams(
            use_tc_tiling_on_sc=False, needs_layout_passes=False),
        out_type=jax.ShapeDtypeStruct((N, D), jnp.float32),
        scratch_types=[
            pltpu.VMEM((P, D), jnp.float32),
            pltpu.VMEM((S, D), jnp.float32),
            pltpu.VMEM((CH,), jnp.int32),
            pltpu.VMEM((CH,), jnp.int32),
            pltpu.VMEM((CH,), jnp.int32),
            pltpu.VMEM((CH,), jnp.int32),
            pltpu.VMEM((CH,), jnp.int32),
            pltpu.VMEM((CH,), jnp.int32),
            pltpu.VMEM((CH, D), jnp.float32),
            pltpu.VMEM((CH, D), jnp.float32),
            pltpu.SemaphoreType.DMA,
            pltpu.SemaphoreType.DMA,
            pltpu.SemaphoreType.DMA,
            pltpu.SemaphoreType.DMA,
            pltpu.SemaphoreType.DMA,
            pltpu.SemaphoreType.DMA,
        ],
    )
    def sc_embed(cat_h, pos_h, shp_h, wtab_h, ptab_h, stab_h, out_h,
                 ptab_v, stab_v,
                 cat0, cat1, pos0, pos1, shp0, shp1, w0b, w1b,
                 sidx0, sidx1, srow0, srow1, sout0, sout1):
        wid = lax.axis_index("s") * NC + lax.axis_index("c")
        base = wid * tok_w
        slots = (
            (cat0, pos0, shp0, w0b, sidx0, srow0, sout0),
            (cat1, pos1, shp1, w1b, sidx1, srow1, sout1),
        )

        pltpu.sync_copy(ptab_h, ptab_v)
        pltpu.sync_copy(stab_h, stab_v)

        def issue_idx(g, sl):
            cat_v, pos_v, shp_v, _, sidx, _, _ = sl
            tok0 = base + g * CH
            pltpu.async_copy(cat_h.at[pl.ds(tok0, CH)], cat_v, sidx)
            pltpu.async_copy(pos_h.at[pl.ds(tok0, CH)], pos_v, sidx)
            pltpu.async_copy(shp_h.at[pl.ds(tok0, CH)], shp_v, sidx)

        def wait_idx(sl):
            cat_v, pos_v, shp_v, _, sidx, _, _ = sl
            pltpu.make_async_copy(cat_h.at[pl.ds(base, CH)], cat_v, sidx).wait()
            pltpu.make_async_copy(pos_h.at[pl.ds(base, CH)], pos_v, sidx).wait()
            pltpu.make_async_copy(shp_h.at[pl.ds(base, CH)], shp_v, sidx).wait()

        def issue_gather(sl):
            cat_v, w_v, srow = sl[0], sl[3], sl[5]
            pltpu.async_copy(wtab_h.at[cat_v], w_v, srow)

        def wait_gather(sl):
            cat_v, w_v, srow = sl[0], sl[3], sl[5]
            pltpu.make_async_copy(wtab_h.at[cat_v], w_v, srow).wait()

        def issue_out(g, sl):
            w_v, sout = sl[3], sl[6]
            tok0 = base + g * CH
            pltpu.async_copy(w_v, out_h.at[pl.ds(tok0, CH)], sout)

        def wait_out(sl):
            w_v, sout = sl[3], sl[6]
            pltpu.make_async_copy(w_v, out_h.at[pl.ds(base, CH)], sout).wait()

        def compute(sl):
            cat_v, pos_v, shp_v, w_v = sl[:4]

            def fix_grp(i, c):
                ids = cat_v[pl.ds(i * _L, _L)]
                msk = ids == 0
                nbad = jnp.max(msk.astype(jnp.int32))

                @pl.when(nbad > 0)
                def _():
                    rows = lax.iota(jnp.int32, _L) + i * _L
                    zeros = jnp.zeros((_L,), jnp.float32)
                    for col in range(D):
                        plsc.store_scatter(
                            w_v, [rows, jnp.full((_L,), col, jnp.int32)],
                            zeros, mask=msk)
                return c

            lax.fori_loop(0, CH // _L, fix_grp, 0)

            def tok_body(t, c):
                tsplat = jnp.full((_L,), 0, jnp.int32) + t
                pid = plsc.load_gather(pos_v, [tsplat])
                sid = plsc.load_gather(shp_v, [tsplat])
                for j in range(D // _L):
                    cols = lax.iota(jnp.int32, _L) + j * _L
                    prow = plsc.load_gather(ptab_v, [pid, cols])
                    srow = plsc.load_gather(stab_v, [sid, cols])
                    sl2 = (t, pl.ds(j * _L, _L))
                    w_v[sl2] = w_v[sl2] + prow + srow
                return c

            lax.fori_loop(0, CH, tok_body, 0)

        issue_idx(0, slots[0])
        issue_idx(1, slots[1])
        wait_idx(slots[0])
        issue_gather(slots[0])

        def outer(go, carry):
            for b in (0, 1):
                g = 2 * go + b
                cur, nxt = slots[b], slots[1 - b]

                @pl.when(g + 1 < n_chunks)
                def _():
                    wait_idx(nxt)

                    @pl.when(g >= 1)
                    def _():
                        wait_out(nxt)

                    issue_gather(nxt)

                wait_gather(cur)

                @pl.when(g + 2 < n_chunks)
                def _():
                    issue_idx(g + 2, cur)

                compute(cur)
                issue_out(g, cur)
            return carry

        lax.fori_loop(0, n_chunks // 2, outer, 0)
        wait_out(slots[0])
        wait_out(slots[1])

    return sc_embed


def kernel(cat_ids, position_ids, shape_ids, word_table, pos_table, shape_table):
    B, L = cat_ids.shape
    V, D = word_table.shape
    N = B * L
    sc_embed = _make_sc_embed(N, D, pos_table.shape[0], shape_table.shape[0])
    out = sc_embed(
        cat_ids.reshape(N),
        position_ids.reshape(N),
        shape_ids.reshape(N),
        word_table,
        pos_table,
        shape_table,
    )
    return out.reshape(B, L, D)
